# Initial kernel scaffold; baseline (speedup 1.0000x reference)
import jax
import jax.numpy as jnp
from jax.experimental import pallas as pl


def kernel(x, W_enc, W_dec, b_dec):
    raise NotImplementedError("write your pallas kernel here")


# speedup vs baseline: 6.6547x; 6.6547x over previous
import jax
import jax.numpy as jnp
from jax.experimental import pallas as pl

_K = 32
_BT = 128


def _enc_body(x_ref, we_ref, hp_ref):
    hp_ref[...] = jax.lax.dot_general(
        x_ref[...], we_ref[...],
        dimension_numbers=(((1,), (1,)), ((), ())),
        preferred_element_type=jnp.float32,
    )


def _topk_body(hp_ref, hs_ref):
    h = hp_ref[...]
    bits = jax.lax.bitcast_convert_type(jnp.abs(h), jnp.int32)

    def step(i, prefix):
        cand = prefix | (jnp.int32(1) << (30 - i))
        cnt = jnp.sum((bits >= cand).astype(jnp.int32), axis=1, keepdims=True)
        return jnp.where(cnt >= _K, cand, prefix)

    thr = jax.lax.fori_loop(
        0, 31, step, jnp.zeros((h.shape[0], 1), jnp.int32), unroll=False
    )
    hs_ref[...] = jnp.where(bits >= thr, h, 0.0)


def _dec_body(hs_ref, wd_ref, b_ref, recon_ref):
    recon_ref[...] = jax.lax.dot_general(
        hs_ref[...], wd_ref[...],
        dimension_numbers=(((1,), (1,)), ((), ())),
        preferred_element_type=jnp.float32,
    ) + b_ref[...]


def kernel(x, W_enc, W_dec, b_dec):
    batch, input_dim = x.shape
    hidden_dim = W_enc.shape[0]
    nb = batch // _BT
    b2 = b_dec.reshape(1, input_dim)

    h_pre = pl.pallas_call(
        _enc_body,
        grid=(nb,),
        in_specs=[
            pl.BlockSpec((_BT, input_dim), lambda i: (i, 0)),
            pl.BlockSpec((hidden_dim, input_dim), lambda i: (0, 0)),
        ],
        out_specs=pl.BlockSpec((_BT, hidden_dim), lambda i: (i, 0)),
        out_shape=jax.ShapeDtypeStruct((batch, hidden_dim), jnp.float32),
    )(x, W_enc)

    h_sparse = pl.pallas_call(
        _topk_body,
        grid=(nb,),
        in_specs=[pl.BlockSpec((_BT, hidden_dim), lambda i: (i, 0))],
        out_specs=pl.BlockSpec((_BT, hidden_dim), lambda i: (i, 0)),
        out_shape=jax.ShapeDtypeStruct((batch, hidden_dim), jnp.float32),
    )(h_pre)

    recon = pl.pallas_call(
        _dec_body,
        grid=(nb,),
        in_specs=[
            pl.BlockSpec((_BT, hidden_dim), lambda i: (i, 0)),
            pl.BlockSpec((input_dim, hidden_dim), lambda i: (0, 0)),
            pl.BlockSpec((1, input_dim), lambda i: (0, 0)),
        ],
        out_specs=pl.BlockSpec((_BT, input_dim), lambda i: (i, 0)),
        out_shape=jax.ShapeDtypeStruct((batch, input_dim), jnp.float32),
    )(h_sparse, W_dec, b2)

    return (recon, h_sparse, h_pre)


# speedup vs baseline: 7.6724x; 1.1529x over previous
import jax
import jax.numpy as jnp
from jax.experimental import pallas as pl
from jax.experimental.pallas import tpu as pltpu

_K = 32
_BT = 128
_BD = 128


def _enc_topk_body(x_ref, we_ref, hp_ref, hs_ref):
    h = jax.lax.dot_general(
        x_ref[...], we_ref[...],
        dimension_numbers=(((1,), (1,)), ((), ())),
        preferred_element_type=jnp.float32,
    )
    hp_ref[...] = h
    hs_ref[...] = jnp.abs(h)

    def step(i, prefix):
        cand = prefix | (jnp.int32(1) << (30 - i))
        candf = jax.lax.bitcast_convert_type(cand, jnp.float32)
        cnt = jnp.sum((hs_ref[...] >= candf).astype(jnp.int32), axis=1,
                      keepdims=True)
        return jnp.where(cnt >= _K, cand, prefix)

    thr = jax.lax.fori_loop(
        0, 31, step, jnp.zeros((h.shape[0], 1), jnp.int32), unroll=False
    )
    thrf = jax.lax.bitcast_convert_type(thr, jnp.float32)
    hs_ref[...] = jnp.where(hs_ref[...] >= thrf, hp_ref[...], 0.0)


def _dec_body(hs_ref, wd_ref, b_ref, recon_ref):
    recon_ref[...] = jax.lax.dot_general(
        hs_ref[...], wd_ref[...],
        dimension_numbers=(((1,), (1,)), ((), ())),
        preferred_element_type=jnp.float32,
    ) + b_ref[...]


def kernel(x, W_enc, W_dec, b_dec):
    batch, input_dim = x.shape
    hidden_dim = W_enc.shape[0]
    nb = batch // _BT
    nd = batch // _BD
    b2 = b_dec.reshape(1, input_dim)

    h_pre, h_sparse = pl.pallas_call(
        _enc_topk_body,
        grid=(nb,),
        in_specs=[
            pl.BlockSpec((_BT, input_dim), lambda i: (i, 0)),
            pl.BlockSpec((hidden_dim, input_dim), lambda i: (0, 0)),
        ],
        out_specs=[
            pl.BlockSpec((_BT, hidden_dim), lambda i: (i, 0)),
            pl.BlockSpec((_BT, hidden_dim), lambda i: (i, 0)),
        ],
        out_shape=[
            jax.ShapeDtypeStruct((batch, hidden_dim), jnp.float32),
            jax.ShapeDtypeStruct((batch, hidden_dim), jnp.float32),
        ],
        compiler_params=pltpu.CompilerParams(
            vmem_limit_bytes=64 * 1024 * 1024,
        ),
    )(x, W_enc)

    recon = pl.pallas_call(
        _dec_body,
        grid=(nd,),
        in_specs=[
            pl.BlockSpec((_BD, hidden_dim), lambda i: (i, 0)),
            pl.BlockSpec((input_dim, hidden_dim), lambda i: (0, 0)),
            pl.BlockSpec((1, input_dim), lambda i: (0, 0)),
        ],
        out_specs=pl.BlockSpec((_BD, input_dim), lambda i: (i, 0)),
        out_shape=jax.ShapeDtypeStruct((batch, input_dim), jnp.float32),
    )(h_sparse, W_dec, b2)

    return (recon, h_sparse, h_pre)
